# baseline (speedup 1.0000x reference)
import functools

import jax
import jax.numpy as jnp
import numpy as np
from jax import lax
from jax.experimental import pallas as pl
from jax.experimental.pallas import tpu as pltpu
from jax.experimental.pallas import tpu_sc as plsc


def _matmul(x, k):
    n, f_in = x.shape
    f_out = k.shape[1]
    bn = 1000
    assert n % bn == 0

    def body(x_ref, k_ref, o_ref):
        o_ref[...] = jnp.dot(x_ref[...], k_ref[...],
                             preferred_element_type=jnp.float32
                             ).astype(jnp.bfloat16)

    return pl.pallas_call(
        body,
        grid=(n // bn,),
        in_specs=[
            pl.BlockSpec((bn, f_in), lambda i: (i, 0)),
            pl.BlockSpec((f_in, f_out), lambda i: (0, 0)),
        ],
        out_specs=pl.BlockSpec((bn, f_out), lambda i: (i, 0)),
        out_shape=jax.ShapeDtypeStruct((n, f_out), jnp.bfloat16),
    )(x, k)


def _epilogue(partials, bias2d):
    nc, n, f = partials.shape
    bn = 1024
    assert n % bn == 0

    def body(p_ref, b_ref, o_ref):
        acc = p_ref[0]
        for c in range(1, nc):
            acc = acc + p_ref[c]
        o_ref[...] = jnp.maximum(acc + b_ref[...], 0.0)

    return pl.pallas_call(
        body,
        grid=(n // bn,),
        in_specs=[
            pl.BlockSpec((nc, bn, f), lambda i: (0, i, 0)),
            pl.BlockSpec((1, f), lambda i: (0, 0)),
        ],
        out_specs=pl.BlockSpec((bn, f), lambda i: (i, 0)),
        out_shape=jax.ShapeDtypeStruct((n, f), jnp.float32),
    )(partials, bias2d)


def _sc_conv(h, wts, src, dst, t, n, f, e):
    info = plsc.get_sparse_core_info()
    nc, ns = info.num_cores, info.num_subcores
    nw = nc * ns
    assert e % nw == 0
    epw = e // nw
    c = 40
    assert epw % c == 0
    nchunk = epw // c
    nb = 4
    nmain = (nchunk // nb) * nb
    assert nchunk - nmain < nb
    pkw = (t + 1) * c
    np_ = ((n + ns * 64 - 1) // (ns * 64)) * (ns * 64)
    rpt = np_ // ns
    assert rpt % c == 0

    mesh = plsc.VectorSubcoreMesh(core_axis_name="c", subcore_axis_name="s")

    @functools.partial(
        pl.kernel,
        out_type=jax.ShapeDtypeStruct((nc, np_, f), jnp.float32),
        mesh=mesh,
        compiler_params=pltpu.CompilerParams(needs_layout_passes=False),
        scratch_types=[
            pltpu.VMEM_SHARED((np_, f), jnp.float32),
            pltpu.VMEM((pkw,), jnp.float32),
            pltpu.VMEM((pkw,), jnp.float32),
            pltpu.VMEM((pkw,), jnp.float32),
            pltpu.VMEM((pkw,), jnp.float32),
            pltpu.VMEM((4, c), jnp.int32),
            pltpu.VMEM((4, c), jnp.int32),
            pltpu.VMEM((4, c, t * f // 2), jnp.int32),
            pltpu.VMEM((c, f), jnp.float32),
            pltpu.SemaphoreType.DMA,
            pltpu.SemaphoreType.DMA,
            pltpu.SemaphoreType.DMA,
            pltpu.SemaphoreType.DMA,
            pltpu.SemaphoreType.DMA,
            pltpu.SemaphoreType.DMA,
            pltpu.SemaphoreType.DMA,
            pltpu.SemaphoreType.DMA,
        ],
    )
    def sck(h_hbm, w_hbm, src_hbm, dst_hbm, out_hbm,
            acc, wbuf0, wbuf1, wbuf2, wbuf3, sidx, didx, rows, msg,
            gs0, gs1, gs2, gs3, ms0, ms1, ms2, ms3):
        cid = lax.axis_index("c")
        sid = lax.axis_index("s")
        wid = sid * nc + cid
        wbufs = (wbuf0, wbuf1, wbuf2, wbuf3)
        gsem = (gs0, gs1, gs2, gs3)
        msem = (ms0, ms1, ms2, ms3)

        zvec = jnp.zeros((16,), jnp.float32)

        def zrow(i, _):
            r = i // (f // 16)
            j = i % (f // 16)
            msg[r, pl.ds(j * 16, 16)] = zvec
            return 0

        lax.fori_loop(0, c * (f // 16), zrow, 0)

        def zcopy(kk, _):
            pltpu.sync_copy(msg, acc.at[pl.ds(sid * rpt + kk * c, c), :])
            return 0

        lax.fori_loop(0, rpt // c, zcopy, 0)
        plsc.subcore_barrier()

        cbase = wid * nchunk
        ebase = wid * epw

        def meta_issue(g, b):
            pltpu.async_copy(w_hbm.at[pl.ds((cbase + g) * pkw, pkw)],
                             wbufs[b], msem[b])
            pltpu.async_copy(src_hbm.at[pl.ds(ebase + g * c, c)],
                             sidx.at[b], msem[b])
            pltpu.async_copy(dst_hbm.at[pl.ds(ebase + g * c, c)],
                             didx.at[b], msem[b])

        def meta_wait(b):
            pltpu.make_async_copy(w_hbm.at[pl.ds(0, pkw)],
                                  wbufs[b], msem[b]).wait()
            pltpu.make_async_copy(src_hbm.at[pl.ds(0, c)],
                                  sidx.at[b], msem[b]).wait()
            pltpu.make_async_copy(dst_hbm.at[pl.ds(0, c)],
                                  didx.at[b], msem[b]).wait()

        def gather_issue(g, b):
            del g
            pltpu.async_copy(h_hbm.at[sidx.at[b]], rows.at[b], gsem[b])

        def gather_wait(b):
            pltpu.make_async_copy(h_hbm.at[sidx.at[b]],
                                  rows.at[b], gsem[b]).wait()

        def process(b):

                def grp(gi, _):
                    e0 = gi * 8
                    wvecs = [wbufs[b][pl.ds(tt * c + e0, 16)]
                             for tt in range(t)]
                    for u in range(8):
                        i = e0 + u
                        ws = [wvecs[tt][u] for tt in range(t)]
                        accs = [None] * (f // 16)
                        for tt in range(t):
                            for j2 in range(f // 32):
                                wv = rows[b, i,
                                          pl.ds(tt * (f // 2) + j2 * 16, 16)]
                                ab = plsc.bitcast(wv, jnp.bfloat16)
                                lo, hi = plsc.unpack(
                                    ab, format=plsc.PackFormat.INTERLEAVED)
                                vl = lo * ws[tt]
                                vh = hi * ws[tt]
                                k2 = j2 * 2
                                if tt == 0:
                                    accs[k2] = vl
                                    accs[k2 + 1] = vh
                                else:
                                    accs[k2] = accs[k2] + vl
                                    accs[k2 + 1] = accs[k2 + 1] + vh
                        for j2 in range(f // 32):
                            msg[i, pl.ds(j2 * 32, 16)] = accs[j2 * 2]
                            msg[i, pl.ds(j2 * 32 + 16, 16)] = accs[j2 * 2 + 1]
                    return 0

                lax.fori_loop(0, c // 8, grp, 0)
                pltpu.sync_copy(msg, acc.at[didx.at[b]], add=True)

        for g0 in range(nb):
            meta_issue(g0, g0)
        for g0 in range(nb - 1):
            meta_wait(g0)
            gather_issue(g0, g0)

        def quad(gq, _):
            for b in range(nb):
                g = gq * nb + b
                la = (b + nb - 1) % nb

                @pl.when(g + nb - 1 < nchunk)
                def _():
                    meta_wait(la)
                    gather_issue(g + nb - 1, la)

                gather_wait(b)
                process(b)

                @pl.when(g + nb < nchunk)
                def _():
                    meta_issue(g + nb, b)

            return 0

        lax.fori_loop(0, nmain // nb, quad, 0)
        for g in range(nmain, nchunk):
            b = g % nb
            gather_wait(b)
            process(b)
        plsc.subcore_barrier()
        pltpu.sync_copy(acc.at[pl.ds(sid * rpt, rpt), :],
                        out_hbm.at[cid, pl.ds(sid * rpt, rpt), :])

    return sck(h, wts, src, dst)


def kernel(node_features, edge_features, indices, out_size, kernel, bias):
    n, f_in = node_features.shape
    t, e = edge_features.shape
    f_out = kernel.shape[2]
    c = 40
    assert f_out % 16 == 0

    k_cat = jnp.transpose(kernel, (1, 0, 2)).reshape(f_in, t * f_out)
    iid = np.arange(t * f_out).reshape(-1, 2, 16)
    perm = np.transpose(iid, (0, 2, 1)).reshape(-1)
    k_cat = jnp.take(k_cat, jnp.asarray(perm), axis=1)
    h = _matmul(node_features, k_cat)
    h = lax.bitcast_convert_type(h.reshape(n, t * f_out // 2, 2), jnp.int32)

    dst = indices[:, 0]
    src = indices[:, 1]
    efc = edge_features.reshape(t, -1, c).transpose(1, 0, 2).reshape(-1, t * c)
    pad = jnp.zeros((e // c, c), jnp.float32)
    wts = jnp.concatenate([efc, pad], axis=1).reshape(-1)

    partials = _sc_conv(h, wts, src, dst, t, n, f_out, e)

    return _epilogue(partials, bias.reshape(1, f_out))[:n]


# speedup vs baseline: 1.0271x; 1.0271x over previous
import functools

import jax
import jax.numpy as jnp
import numpy as np
from jax import lax
from jax.experimental import pallas as pl
from jax.experimental.pallas import tpu as pltpu
from jax.experimental.pallas import tpu_sc as plsc


def _matmul(x, k):
    n, f_in = x.shape
    f_out = k.shape[1]
    bn = 1000
    assert n % bn == 0

    def body(x_ref, k_ref, o_ref):
        o_ref[...] = jnp.dot(x_ref[...], k_ref[...],
                             preferred_element_type=jnp.float32
                             ).astype(jnp.bfloat16)

    return pl.pallas_call(
        body,
        grid=(n // bn,),
        in_specs=[
            pl.BlockSpec((bn, f_in), lambda i: (i, 0)),
            pl.BlockSpec((f_in, f_out), lambda i: (0, 0)),
        ],
        out_specs=pl.BlockSpec((bn, f_out), lambda i: (i, 0)),
        out_shape=jax.ShapeDtypeStruct((n, f_out), jnp.bfloat16),
    )(x, k)


def _epilogue(partials, bias2d):
    nc, n, f = partials.shape
    bn = 1024
    assert n % bn == 0

    def body(p_ref, b_ref, o_ref):
        acc = p_ref[0]
        for c in range(1, nc):
            acc = acc + p_ref[c]
        o_ref[...] = jnp.maximum(acc + b_ref[...], 0.0)

    return pl.pallas_call(
        body,
        grid=(n // bn,),
        in_specs=[
            pl.BlockSpec((nc, bn, f), lambda i: (0, i, 0)),
            pl.BlockSpec((1, f), lambda i: (0, 0)),
        ],
        out_specs=pl.BlockSpec((bn, f), lambda i: (i, 0)),
        out_shape=jax.ShapeDtypeStruct((n, f), jnp.float32),
    )(partials, bias2d)


def _sc_conv(h, wts, src, dst, t, n, f, e):
    info = plsc.get_sparse_core_info()
    nc, ns = info.num_cores, info.num_subcores
    nw = nc * ns
    assert e % nw == 0
    epw = e // nw
    c = 40
    assert epw % c == 0
    nchunk = epw // c
    assert nchunk % 2 == 0
    pkw = (t + 1) * c
    np_ = ((n + ns * 64 - 1) // (ns * 64)) * (ns * 64)
    rpt = np_ // ns
    assert rpt % c == 0

    mesh = plsc.VectorSubcoreMesh(core_axis_name="c", subcore_axis_name="s")

    @functools.partial(
        pl.kernel,
        out_type=jax.ShapeDtypeStruct((nc, np_, f), jnp.float32),
        mesh=mesh,
        compiler_params=pltpu.CompilerParams(needs_layout_passes=False),
        scratch_types=[
            pltpu.VMEM_SHARED((np_, f), jnp.float32),
            pltpu.VMEM((pkw,), jnp.float32),
            pltpu.VMEM((pkw,), jnp.float32),
            pltpu.VMEM((2, c), jnp.int32),
            pltpu.VMEM((2, c), jnp.int32),
            pltpu.VMEM((2, c, t * f // 2), jnp.int32),
            pltpu.VMEM((c, f), jnp.float32),
            pltpu.SemaphoreType.DMA,
            pltpu.SemaphoreType.DMA,
            pltpu.SemaphoreType.DMA,
            pltpu.SemaphoreType.DMA,
        ],
    )
    def sck(h_hbm, w_hbm, src_hbm, dst_hbm, out_hbm,
            acc, wbuf0, wbuf1, sidx, didx, rows, msg, gs0, gs1, ms0, ms1):
        cid = lax.axis_index("c")
        sid = lax.axis_index("s")
        wid = sid * nc + cid
        wbufs = (wbuf0, wbuf1)
        gsem = (gs0, gs1)
        msem = (ms0, ms1)

        zvec = jnp.zeros((16,), jnp.float32)

        def zrow(i, _):
            r = i // (f // 16)
            j = i % (f // 16)
            msg[r, pl.ds(j * 16, 16)] = zvec
            return 0

        lax.fori_loop(0, c * (f // 16), zrow, 0)

        def zcopy(kk, _):
            pltpu.sync_copy(msg, acc.at[pl.ds(sid * rpt + kk * c, c), :])
            return 0

        lax.fori_loop(0, rpt // c, zcopy, 0)
        plsc.subcore_barrier()

        cbase = wid * nchunk
        ebase = wid * epw

        def meta_issue(g, b):
            pltpu.async_copy(w_hbm.at[pl.ds((cbase + g) * pkw, pkw)],
                             wbufs[b], msem[b])
            pltpu.async_copy(src_hbm.at[pl.ds(ebase + g * c, c)],
                             sidx.at[b], msem[b])
            pltpu.async_copy(dst_hbm.at[pl.ds(ebase + g * c, c)],
                             didx.at[b], msem[b])

        def meta_wait(b):
            pltpu.make_async_copy(w_hbm.at[pl.ds(0, pkw)],
                                  wbufs[b], msem[b]).wait()
            pltpu.make_async_copy(src_hbm.at[pl.ds(0, c)],
                                  sidx.at[b], msem[b]).wait()
            pltpu.make_async_copy(dst_hbm.at[pl.ds(0, c)],
                                  didx.at[b], msem[b]).wait()

        def gather_issue(g, b):
            del g
            pltpu.async_copy(h_hbm.at[sidx.at[b]], rows.at[b], gsem[b])

        def gather_wait(b):
            pltpu.make_async_copy(h_hbm.at[sidx.at[b]],
                                  rows.at[b], gsem[b]).wait()

        meta_issue(0, 0)
        meta_wait(0)
        gather_issue(0, 0)
        meta_issue(1, 1)

        def pair(gg, _):
            for b in range(2):
                g = gg * 2 + b
                nb = 1 - b

                @pl.when(g + 1 < nchunk)
                def _():
                    meta_wait(nb)
                    gather_issue(g + 1, nb)

                gather_wait(b)

                def grp(gi, _):
                    e0 = gi * 8
                    wvecs = [wbufs[b][pl.ds(tt * c + e0, 16)]
                             for tt in range(t)]
                    for u in range(8):
                        i = e0 + u
                        ws = [wvecs[tt][u] for tt in range(t)]
                        accs = [None] * (f // 16)
                        for tt in range(t):
                            for j2 in range(f // 32):
                                wv = rows[b, i,
                                          pl.ds(tt * (f // 2) + j2 * 16, 16)]
                                ab = plsc.bitcast(wv, jnp.bfloat16)
                                lo, hi = plsc.unpack(
                                    ab, format=plsc.PackFormat.INTERLEAVED)
                                vl = lo * ws[tt]
                                vh = hi * ws[tt]
                                k2 = j2 * 2
                                if tt == 0:
                                    accs[k2] = vl
                                    accs[k2 + 1] = vh
                                else:
                                    accs[k2] = accs[k2] + vl
                                    accs[k2 + 1] = accs[k2 + 1] + vh
                        for j2 in range(f // 32):
                            msg[i, pl.ds(j2 * 32, 16)] = accs[j2 * 2]
                            msg[i, pl.ds(j2 * 32 + 16, 16)] = accs[j2 * 2 + 1]
                    return 0

                lax.fori_loop(0, c // 8, grp, 0)
                pltpu.sync_copy(msg, acc.at[didx.at[b]], add=True)

                @pl.when(g + 2 < nchunk)
                def _():
                    meta_issue(g + 2, b)

            return 0

        lax.fori_loop(0, nchunk // 2, pair, 0)
        plsc.subcore_barrier()
        pltpu.sync_copy(acc.at[pl.ds(sid * rpt, rpt), :],
                        out_hbm.at[cid, pl.ds(sid * rpt, rpt), :])

    return sck(h, wts, src, dst)


def kernel(node_features, edge_features, indices, out_size, kernel, bias):
    n, f_in = node_features.shape
    t, e = edge_features.shape
    f_out = kernel.shape[2]
    c = 40
    assert f_out % 16 == 0

    k_cat = jnp.transpose(kernel, (1, 0, 2)).reshape(f_in, t * f_out)
    iid = np.arange(t * f_out).reshape(-1, 2, 16)
    perm = np.transpose(iid, (0, 2, 1)).reshape(-1)
    k_cat = jnp.take(k_cat, jnp.asarray(perm), axis=1)
    h = _matmul(node_features, k_cat)
    h = lax.bitcast_convert_type(h.reshape(n, t * f_out // 2, 2), jnp.int32)

    dst = indices[:, 0]
    src = indices[:, 1]
    efc = edge_features.reshape(t, -1, c).transpose(1, 0, 2).reshape(-1, t * c)
    pad = jnp.zeros((e // c, c), jnp.float32)
    wts = jnp.concatenate([efc, pad], axis=1).reshape(-1)

    partials = _sc_conv(h, wts, src, dst, t, n, f_out, e)

    return _epilogue(partials, bias.reshape(1, f_out))[:n]


# speedup vs baseline: 1.1033x; 1.0742x over previous
import functools

import jax
import jax.numpy as jnp
import numpy as np
from jax import lax
from jax.experimental import pallas as pl
from jax.experimental.pallas import tpu as pltpu
from jax.experimental.pallas import tpu_sc as plsc


def _matmul(x, k):
    n, f_in = x.shape
    f_out = k.shape[1]
    bn = 1000
    assert n % bn == 0

    def body(x_ref, k_ref, o_ref):
        o_ref[...] = jnp.dot(x_ref[...], k_ref[...],
                             preferred_element_type=jnp.float32
                             ).astype(jnp.bfloat16)

    return pl.pallas_call(
        body,
        grid=(n // bn,),
        in_specs=[
            pl.BlockSpec((bn, f_in), lambda i: (i, 0)),
            pl.BlockSpec((f_in, f_out), lambda i: (0, 0)),
        ],
        out_specs=pl.BlockSpec((bn, f_out), lambda i: (i, 0)),
        out_shape=jax.ShapeDtypeStruct((n, f_out), jnp.bfloat16),
    )(x, k)


def _epilogue(partials, bias2d):
    nc, n, f = partials.shape
    bn = 1024
    assert n % bn == 0

    def body(p_ref, b_ref, o_ref):
        acc = p_ref[0]
        for c in range(1, nc):
            acc = acc + p_ref[c]
        o_ref[...] = jnp.maximum(acc + b_ref[...], 0.0)

    return pl.pallas_call(
        body,
        grid=(n // bn,),
        in_specs=[
            pl.BlockSpec((nc, bn, f), lambda i: (0, i, 0)),
            pl.BlockSpec((1, f), lambda i: (0, 0)),
        ],
        out_specs=pl.BlockSpec((bn, f), lambda i: (i, 0)),
        out_shape=jax.ShapeDtypeStruct((n, f), jnp.float32),
    )(partials, bias2d)


def _sc_conv(h, wts, src, dst, t, n, f, e):
    info = plsc.get_sparse_core_info()
    nc, ns = info.num_cores, info.num_subcores
    nw = nc * ns
    assert e % nw == 0
    epw = e // nw
    c = 80
    cm = c // 2
    assert epw % c == 0
    nchunk = epw // c
    nmain = (nchunk // 2) * 2
    pkw = (t + 1) * c
    np_ = ((n + ns * 64 - 1) // (ns * 64)) * (ns * 64)
    rpt = np_ // ns
    assert rpt % cm == 0

    mesh = plsc.VectorSubcoreMesh(core_axis_name="c", subcore_axis_name="s")

    @functools.partial(
        pl.kernel,
        out_type=jax.ShapeDtypeStruct((nc, np_, f), jnp.float32),
        mesh=mesh,
        compiler_params=pltpu.CompilerParams(needs_layout_passes=False),
        scratch_types=[
            pltpu.VMEM_SHARED((np_, f), jnp.float32),
            pltpu.VMEM((pkw,), jnp.float32),
            pltpu.VMEM((pkw,), jnp.float32),
            pltpu.VMEM((2, c), jnp.int32),
            pltpu.VMEM((2, 2, cm), jnp.int32),
            pltpu.VMEM((2, c, t * f // 2), jnp.int32),
            pltpu.VMEM((cm, f), jnp.float32),
            pltpu.SemaphoreType.DMA,
            pltpu.SemaphoreType.DMA,
            pltpu.SemaphoreType.DMA,
            pltpu.SemaphoreType.DMA,
        ],
    )
    def sck(h_hbm, w_hbm, src_hbm, dst_hbm, out_hbm,
            acc, wbuf0, wbuf1, sidx, didx, rows, msg, gs0, gs1, ms0, ms1):
        cid = lax.axis_index("c")
        sid = lax.axis_index("s")
        wid = sid * nc + cid
        wbufs = (wbuf0, wbuf1)
        gsem = (gs0, gs1)
        msem = (ms0, ms1)

        zvec = jnp.zeros((16,), jnp.float32)

        def zrow(i, _):
            r = i // (f // 16)
            j = i % (f // 16)
            msg[r, pl.ds(j * 16, 16)] = zvec
            return 0

        lax.fori_loop(0, cm * (f // 16), zrow, 0)

        def zcopy(kk, _):
            pltpu.sync_copy(msg, acc.at[pl.ds(sid * rpt + kk * cm, cm), :])
            return 0

        lax.fori_loop(0, rpt // cm, zcopy, 0)
        plsc.subcore_barrier()

        cbase = wid * nchunk
        ebase = wid * epw

        def meta_issue(g, b):
            pltpu.async_copy(w_hbm.at[pl.ds((cbase + g) * pkw, pkw)],
                             wbufs[b], msem[b])
            pltpu.async_copy(src_hbm.at[pl.ds(ebase + g * c, c)],
                             sidx.at[b], msem[b])
            for hh in range(2):
                pltpu.async_copy(
                    dst_hbm.at[pl.ds(ebase + g * c + hh * cm, cm)],
                    didx.at[b, hh], msem[b])

        def meta_wait(b):
            pltpu.make_async_copy(w_hbm.at[pl.ds(0, pkw)],
                                  wbufs[b], msem[b]).wait()
            pltpu.make_async_copy(src_hbm.at[pl.ds(0, c)],
                                  sidx.at[b], msem[b]).wait()
            for hh in range(2):
                pltpu.make_async_copy(dst_hbm.at[pl.ds(0, cm)],
                                      didx.at[b, hh], msem[b]).wait()

        def gather_issue(g, b):
            del g
            pltpu.async_copy(h_hbm.at[sidx.at[b]], rows.at[b], gsem[b])

        def gather_wait(b):
            pltpu.make_async_copy(h_hbm.at[sidx.at[b]],
                                  rows.at[b], gsem[b]).wait()

        meta_issue(0, 0)
        meta_wait(0)
        gather_issue(0, 0)
        meta_issue(1, 1)

        def pair(gg, _):
            for b in range(2):
                g = gg * 2 + b
                nb = 1 - b

                @pl.when(g + 1 < nchunk)
                def _():
                    meta_wait(nb)
                    gather_issue(g + 1, nb)

                gather_wait(b)
                process(b)

                @pl.when(g + 2 < nchunk)
                def _():
                    meta_issue(g + 2, b)

            return 0

        def process(b):
            for hh in range(2):
                def grp(gi, _):
                    e0 = gi * 8
                    wvecs = [wbufs[b][pl.ds(tt * c + hh * cm + e0, 16)]
                             for tt in range(t)]
                    for u in range(8):
                        mi = e0 + u
                        i = hh * cm + mi
                        ws = [wvecs[tt][u] for tt in range(t)]
                        accs = [None] * (f // 16)
                        for tt in range(t):
                            for j2 in range(f // 32):
                                wv = rows[b, i,
                                          pl.ds(tt * (f // 2) + j2 * 16, 16)]
                                ab = plsc.bitcast(wv, jnp.bfloat16)
                                lo, hi = plsc.unpack(
                                    ab, format=plsc.PackFormat.INTERLEAVED)
                                vl = lo * ws[tt]
                                vh = hi * ws[tt]
                                k2 = j2 * 2
                                if tt == 0:
                                    accs[k2] = vl
                                    accs[k2 + 1] = vh
                                else:
                                    accs[k2] = accs[k2] + vl
                                    accs[k2 + 1] = accs[k2 + 1] + vh
                        for j2 in range(f // 32):
                            msg[mi, pl.ds(j2 * 32, 16)] = accs[j2 * 2]
                            msg[mi, pl.ds(j2 * 32 + 16, 16)] = \
                                accs[j2 * 2 + 1]
                    return 0

                lax.fori_loop(0, cm // 8, grp, 0)
                pltpu.sync_copy(msg, acc.at[didx.at[b, hh]], add=True)

        lax.fori_loop(0, nmain // 2, pair, 0)
        for g in range(nmain, nchunk):
            gather_wait(g % 2)
            process(g % 2)
        plsc.subcore_barrier()
        pltpu.sync_copy(acc.at[pl.ds(sid * rpt, rpt), :],
                        out_hbm.at[cid, pl.ds(sid * rpt, rpt), :])

    return sck(h, wts, src, dst)


def kernel(node_features, edge_features, indices, out_size, kernel, bias):
    n, f_in = node_features.shape
    t, e = edge_features.shape
    f_out = kernel.shape[2]
    c = 80
    assert f_out % 16 == 0

    k_cat = jnp.transpose(kernel, (1, 0, 2)).reshape(f_in, t * f_out)
    iid = np.arange(t * f_out).reshape(-1, 2, 16)
    perm = np.transpose(iid, (0, 2, 1)).reshape(-1)
    k_cat = jnp.take(k_cat, jnp.asarray(perm), axis=1)
    h = _matmul(node_features, k_cat)
    h = lax.bitcast_convert_type(h.reshape(n, t * f_out // 2, 2), jnp.int32)

    dst = indices[:, 0]
    src = indices[:, 1]
    efc = edge_features.reshape(t, -1, c).transpose(1, 0, 2).reshape(-1, t * c)
    pad = jnp.zeros((e // c, c), jnp.float32)
    wts = jnp.concatenate([efc, pad], axis=1).reshape(-1)

    partials = _sc_conv(h, wts, src, dst, t, n, f_out, e)

    return _epilogue(partials, bias.reshape(1, f_out))[:n]
